# baseline (speedup 1.0000x reference)
import functools

import jax
import jax.numpy as jnp
from jax import lax
from jax.experimental import pallas as pl
from jax.experimental.pallas import tpu as pltpu
from jax.experimental.pallas import tpu_sc as plsc

N_GENES = 30000
N_DIM = 64
N_CELLS = 512
N_SEL = 2000

_NC = 2
_NS = 16
_NW = _NC * _NS
_SEL_PAD = 2048
_B_PER_W = _SEL_PAD // _NW


def _sc_gather_body(table_hbm, idx_hbm, bias_hbm, w_out, b_out,
                    idx_v, rows_v, bvals_v, sem, bsem):
    wid = lax.axis_index("s") * _NC + lax.axis_index("c")
    base = wid * _B_PER_W
    pltpu.sync_copy(idx_hbm.at[pl.ds(base, _B_PER_W)], idx_v)
    wcopy = pltpu.async_copy(table_hbm.at[idx_v], rows_v, sem)
    bcopy = pltpu.async_copy(bias_hbm.at[idx_v], bvals_v, bsem)
    wcopy.wait()
    pltpu.sync_copy(rows_v, w_out.at[pl.ds(base, _B_PER_W)])
    bcopy.wait()
    pltpu.sync_copy(bvals_v, b_out.at[pl.ds(base, _B_PER_W)])


def _sc_gather(weight1, idx_pad, bias1):
    mesh = plsc.VectorSubcoreMesh(core_axis_name="c", subcore_axis_name="s")
    k = functools.partial(
        pl.kernel,
        mesh=mesh,
        out_type=(
            jax.ShapeDtypeStruct((_SEL_PAD, N_DIM), jnp.float32),
            jax.ShapeDtypeStruct((_SEL_PAD,), jnp.float32),
        ),
        scratch_types=[
            pltpu.VMEM((_B_PER_W,), jnp.int32),
            pltpu.VMEM((_B_PER_W, N_DIM), jnp.float32),
            pltpu.VMEM((_B_PER_W,), jnp.float32),
            pltpu.SemaphoreType.DMA,
            pltpu.SemaphoreType.DMA,
        ],
        compiler_params=pltpu.CompilerParams(use_tc_tiling_on_sc=False),
    )(_sc_gather_body)
    return k(weight1, idx_pad, bias1)


_S_CH = 80
_N_CH = N_SEL // _S_CH
_C_PER_W = N_CELLS // _NW
_C_HALF = _C_PER_W // 2
_L = 16


def _sc_compute_body(emb_hbm, w_hbm, b_hbm, out_hbm,
                     emb_v, w_v, b_v, outc_v, sem, wsem):
    wid = lax.axis_index("s") * _NC + lax.axis_index("c")
    pltpu.sync_copy(b_hbm, b_v)

    for p in range(2):
        cbase = wid * _C_PER_W + p * _C_HALF

        def chunk_copies(k, slot):
            ops = [pltpu.make_async_copy(
                w_hbm.at[pl.ds(k * _S_CH, _S_CH), :],
                w_v.at[slot], sem.at[slot])]
            for ci in range(_C_HALF):
                ops.append(pltpu.make_async_copy(
                    emb_hbm.at[cbase + ci, pl.ds(k * _S_CH, _S_CH), :],
                    emb_v.at[slot, ci], sem.at[slot]))
            return ops

        def out_copies(k, slot):
            return [pltpu.make_async_copy(
                outc_v.at[slot, ci],
                out_hbm.at[cbase + ci, pl.ds(k * _S_CH, _S_CH)],
                wsem.at[slot]) for ci in range(_C_HALF)]

        for op in chunk_copies(0, 0):
            op.start()

        def k_body(k, carry):
            slot = lax.rem(k, 2)

            @pl.when(k + 1 < _N_CH)
            def _():
                for op in chunk_copies(k + 1, 1 - slot):
                    op.start()

            for op in chunk_copies(k, slot):
                op.wait()

            @pl.when(k >= 2)
            def _():
                for op in out_copies(k - 2, slot):
                    op.wait()

            lane = lax.broadcasted_iota(jnp.int32, (_L,), 0)

            def sg_body(sg, c2):
                loc = sg * _L
                bias = b_v[pl.ds(k * _S_CH + loc, _L)]
                outs = [jnp.zeros((_L,), jnp.float32)
                        for _ in range(_C_HALF)]
                for s in range(_L):
                    srow = loc + s
                    wvs = [w_v[slot, srow, pl.ds(g * _L, _L)]
                           for g in range(N_DIM // _L)]
                    for ci in range(_C_HALF):
                        acc = (emb_v[slot, ci, srow, pl.ds(0, _L)]
                               * wvs[0])
                        for g in range(1, N_DIM // _L):
                            acc = acc + (
                                emb_v[slot, ci, srow, pl.ds(g * _L, _L)]
                                * wvs[g])
                        outs[ci] = jnp.where(lane == s,
                                             jnp.sum(acc, axis=0),
                                             outs[ci])
                for ci in range(_C_HALF):
                    outc_v[slot, ci, pl.ds(loc, _L)] = outs[ci] + bias
                return c2

            lax.fori_loop(0, _S_CH // _L, sg_body, 0)

            for op in out_copies(k, slot):
                op.start()
            return carry

        lax.fori_loop(0, _N_CH, k_body, 0)
        for kk in (_N_CH - 2, _N_CH - 1):
            for op in out_copies(kk, kk % 2):
                op.wait()


def _sc_compute(emb, w, b):
    mesh = plsc.VectorSubcoreMesh(core_axis_name="c", subcore_axis_name="s")
    k = functools.partial(
        pl.kernel,
        mesh=mesh,
        out_type=jax.ShapeDtypeStruct((N_CELLS, N_SEL), jnp.float32),
        scratch_types=[
            pltpu.VMEM((2, _C_HALF, _S_CH, N_DIM), jnp.float32),
            pltpu.VMEM((2, _S_CH, N_DIM), jnp.float32),
            pltpu.VMEM((N_SEL,), jnp.float32),
            pltpu.VMEM((2, _C_HALF, _S_CH), jnp.float32),
            pltpu.SemaphoreType.DMA((2,)),
            pltpu.SemaphoreType.DMA((2,)),
        ],
        compiler_params=pltpu.CompilerParams(
            use_tc_tiling_on_sc=False, needs_layout_passes=False),
    )(_sc_compute_body)
    return k(emb, w, b)


def kernel(cell_gene_embedding, gene_ix, weight1, bias1):
    idx_pad = jnp.zeros((_SEL_PAD,), jnp.int32).at[:N_SEL].set(
        gene_ix.astype(jnp.int32))
    w_sel, b_sel = _sc_gather(weight1, idx_pad, bias1)
    return _sc_compute(cell_gene_embedding, w_sel[:N_SEL], b_sel[:N_SEL])


# speedup vs baseline: 1.2648x; 1.2648x over previous
import functools

import jax
import jax.numpy as jnp
from jax import lax
from jax.experimental import pallas as pl
from jax.experimental.pallas import tpu as pltpu
from jax.experimental.pallas import tpu_sc as plsc

N_GENES = 30000
N_DIM = 64
N_CELLS = 512
N_SEL = 2000

_NC = 2
_NS = 16
_NW = _NC * _NS
_SEL_PAD = 2048
_B_PER_W = _SEL_PAD // _NW


def _sc_gather_body(table_hbm, idx_hbm, bias_hbm, w_out, b_out,
                    idx_v, rows_v, bvals_v, sem, bsem):
    wid = lax.axis_index("s") * _NC + lax.axis_index("c")
    base = wid * _B_PER_W
    pltpu.sync_copy(idx_hbm.at[pl.ds(base, _B_PER_W)], idx_v)
    wcopy = pltpu.async_copy(table_hbm.at[idx_v], rows_v, sem)
    bcopy = pltpu.async_copy(bias_hbm.at[idx_v], bvals_v, bsem)
    wcopy.wait()
    pltpu.sync_copy(rows_v, w_out.at[pl.ds(base, _B_PER_W)])
    bcopy.wait()
    pltpu.sync_copy(bvals_v, b_out.at[pl.ds(base, _B_PER_W)])


def _sc_gather(weight1, idx_pad, bias1):
    mesh = plsc.VectorSubcoreMesh(core_axis_name="c", subcore_axis_name="s")
    k = functools.partial(
        pl.kernel,
        mesh=mesh,
        out_type=(
            jax.ShapeDtypeStruct((_SEL_PAD, N_DIM), jnp.float32),
            jax.ShapeDtypeStruct((_SEL_PAD,), jnp.float32),
        ),
        scratch_types=[
            pltpu.VMEM((_B_PER_W,), jnp.int32),
            pltpu.VMEM((_B_PER_W, N_DIM), jnp.float32),
            pltpu.VMEM((_B_PER_W,), jnp.float32),
            pltpu.SemaphoreType.DMA,
            pltpu.SemaphoreType.DMA,
        ],
        compiler_params=pltpu.CompilerParams(use_tc_tiling_on_sc=False),
    )(_sc_gather_body)
    return k(weight1, idx_pad, bias1)


_C_BLK = 8
_N_STEP = N_CELLS // _C_BLK
_M_BLK = _C_BLK * N_SEL
_NBUF = 4


def _tc_body(emb_hbm, w_ref, b_ref, p_ref, d_ref, out_ref, buf, sem):
    i = pl.program_id(0)
    emb_r = emb_hbm.reshape(N_CELLS * N_SEL, N_DIM)

    def start(blk):
        slot = lax.rem(blk, _NBUF)
        pltpu.make_async_copy(
            emb_r.at[pl.ds(blk * _M_BLK, _M_BLK)],
            buf.at[slot], sem.at[slot]).start()

    @pl.when(i == 0)
    def _():
        for b in range(_NBUF - 1):
            start(b)

    @pl.when(i + (_NBUF - 1) < _N_STEP)
    def _():
        start(i + (_NBUF - 1))

    slot = lax.rem(i, _NBUF)
    pltpu.make_async_copy(
        emb_r.at[pl.ds(i * _M_BLK, _M_BLK)],
        buf.at[slot], sem.at[slot]).wait()

    x3 = buf.at[slot].reshape(_C_BLK, N_SEL, N_DIM)
    for g in range(16):
        sg = 128 if g < 15 else N_SEL - 15 * 128
        rows = pl.ds(g * 128, sg)
        chunk = x3[:, rows, :] * w_ref[rows, :][None]
        a2 = chunk.reshape(_C_BLK * sg, N_DIM)
        z = jnp.dot(a2, p_ref[...], preferred_element_type=jnp.float32)
        z3 = z.reshape(_C_BLK, sg, 128)
        out_g = jnp.sum(z3 * d_ref[:sg, :][None], axis=1)
        cols = pl.ds(g * 128, sg)
        out_ref[:, cols] = out_g[:, :sg] + b_ref[0, cols][None, :]


def _tc_dense(emb, w, b2d, pmat, dmask):
    return pl.pallas_call(
        _tc_body,
        grid=(_N_STEP,),
        in_specs=[
            pl.BlockSpec(memory_space=pltpu.MemorySpace.HBM),
            pl.BlockSpec((N_SEL, N_DIM), lambda i: (0, 0)),
            pl.BlockSpec((1, N_SEL), lambda i: (0, 0)),
            pl.BlockSpec((N_DIM, 128), lambda i: (0, 0)),
            pl.BlockSpec((128, 128), lambda i: (0, 0)),
        ],
        out_specs=pl.BlockSpec((_C_BLK, N_SEL), lambda i: (i, 0)),
        out_shape=jax.ShapeDtypeStruct((N_CELLS, N_SEL), jnp.float32),
        scratch_shapes=[
            pltpu.VMEM((_NBUF, _M_BLK, N_DIM), jnp.float32),
            pltpu.SemaphoreType.DMA((_NBUF,)),
        ],
    )(emb, w, b2d, pmat, dmask)


def kernel(cell_gene_embedding, gene_ix, weight1, bias1):
    idx_pad = jnp.zeros((_SEL_PAD,), jnp.int32).at[:N_SEL].set(
        gene_ix.astype(jnp.int32))
    w_sel, b_sel = _sc_gather(weight1, idx_pad, bias1)
    b2d = b_sel[:N_SEL].reshape(1, N_SEL)
    pmat = jnp.ones((N_DIM, 128), jnp.float32)
    lane = jnp.arange(128, dtype=jnp.int32)
    dmask = (lane[:, None] == lane[None, :]).astype(jnp.float32)
    return _tc_dense(cell_gene_embedding, w_sel[:N_SEL], b2d, pmat, dmask)


# speedup vs baseline: 2.1795x; 1.7232x over previous
import functools

import jax
import jax.numpy as jnp
from jax import lax
from jax.experimental import pallas as pl
from jax.experimental.pallas import tpu as pltpu
from jax.experimental.pallas import tpu_sc as plsc

N_GENES = 30000
N_DIM = 64
N_CELLS = 512
N_SEL = 2000

_NC = 2
_NS = 16
_NW = _NC * _NS
_SEL_PAD = 2048
_B_PER_W = _SEL_PAD // _NW


def _sc_gather_body(table_hbm, idx_hbm, bias_hbm, w_out, b_out,
                    idx_v, rows_v, bvals_v, sem, bsem):
    wid = lax.axis_index("s") * _NC + lax.axis_index("c")
    base = wid * _B_PER_W
    pltpu.sync_copy(idx_hbm.at[pl.ds(base, _B_PER_W)], idx_v)
    wcopy = pltpu.async_copy(table_hbm.at[idx_v], rows_v, sem)
    bcopy = pltpu.async_copy(bias_hbm.at[idx_v], bvals_v, bsem)
    wcopy.wait()
    pltpu.sync_copy(rows_v, w_out.at[pl.ds(base, _B_PER_W)])
    bcopy.wait()
    pltpu.sync_copy(bvals_v, b_out.at[pl.ds(base, _B_PER_W)])


def _sc_gather(weight1, idx_pad, bias1):
    mesh = plsc.VectorSubcoreMesh(core_axis_name="c", subcore_axis_name="s")
    k = functools.partial(
        pl.kernel,
        mesh=mesh,
        out_type=(
            jax.ShapeDtypeStruct((_SEL_PAD, N_DIM), jnp.float32),
            jax.ShapeDtypeStruct((_SEL_PAD,), jnp.float32),
        ),
        scratch_types=[
            pltpu.VMEM((_B_PER_W,), jnp.int32),
            pltpu.VMEM((_B_PER_W, N_DIM), jnp.float32),
            pltpu.VMEM((_B_PER_W,), jnp.float32),
            pltpu.SemaphoreType.DMA,
            pltpu.SemaphoreType.DMA,
        ],
        compiler_params=pltpu.CompilerParams(use_tc_tiling_on_sc=False),
    )(_sc_gather_body)
    return k(weight1, idx_pad, bias1)


_T = N_SEL // 2
_C_BLK = 32
_N_STEP = N_CELLS // _C_BLK
_TG = 64
_N_GRP = 16


def _tc_body(emb_ref, wp_ref, b_ref, p_ref, d_ref, out_ref):
    for g in range(_N_GRP):
        tg = _TG if g < _N_GRP - 1 else _T - (_N_GRP - 1) * _TG
        sg = 2 * tg
        rows = pl.ds(g * _TG, tg)
        chunk = emb_ref[:, rows, :] * wp_ref[:, rows, :]
        a2 = chunk.reshape(_C_BLK * tg, 128)
        z = jnp.dot(a2, p_ref[...], preferred_element_type=jnp.float32)
        z3 = z.reshape(_C_BLK, tg, 128)
        out_g = jnp.sum(z3 * d_ref[:tg, :][None], axis=1)
        cols = pl.ds(g * 2 * _TG, sg)
        out_ref[:, cols] = out_g[:, :sg] + b_ref[0, cols][None, :]


def _tc_dense(emb13, wp, b2d, pmat, dmask):
    return pl.pallas_call(
        _tc_body,
        grid=(_N_STEP,),
        in_specs=[
            pl.BlockSpec((_C_BLK, _T, 128), lambda i: (i, 0, 0)),
            pl.BlockSpec((1, _T, 128), lambda i: (0, 0, 0)),
            pl.BlockSpec((1, N_SEL), lambda i: (0, 0)),
            pl.BlockSpec((128, 128), lambda i: (0, 0)),
            pl.BlockSpec((_TG, 128), lambda i: (0, 0)),
        ],
        out_specs=pl.BlockSpec((_C_BLK, N_SEL), lambda i: (i, 0)),
        out_shape=jax.ShapeDtypeStruct((N_CELLS, N_SEL), jnp.float32),
    )(emb13, wp, b2d, pmat, dmask)


def kernel(cell_gene_embedding, gene_ix, weight1, bias1):
    idx_pad = jnp.zeros((_SEL_PAD,), jnp.int32).at[:N_SEL].set(
        gene_ix.astype(jnp.int32))
    w_sel, b_sel = _sc_gather(weight1, idx_pad, bias1)
    wp = w_sel[:N_SEL].reshape(1, _T, 128)
    b2d = b_sel[:N_SEL].reshape(1, N_SEL)
    emb13 = cell_gene_embedding.reshape(N_CELLS, _T, 128)
    lane = jnp.arange(128, dtype=jnp.int32)
    pmat = ((lane[:, None] < 64) == (lane[None, :] % 2 == 0)
            ).astype(jnp.float32)
    trow = jnp.arange(_TG, dtype=jnp.int32)
    dmask = (trow[:, None] == lane[None, :] // 2).astype(jnp.float32)
    return _tc_dense(emb13, wp, b2d, pmat, dmask)
